# Initial kernel scaffold; baseline (speedup 1.0000x reference)
import jax
import jax.numpy as jnp
from jax.experimental import pallas as pl


def kernel(x, adj, rep_adj_dis, Wqkv):
    raise NotImplementedError("write your pallas kernel here")


# speedup vs baseline: 90.2403x; 90.2403x over previous
import functools

import jax
import jax.numpy as jnp
import numpy as np
from jax.experimental import pallas as pl
from jax.experimental.pallas import tpu as pltpu

_DIM = 768
_HEADS = 12
_HD = _DIM // _HEADS
_TOPK = 64
_T = 256

_HIGHEST = jax.lax.Precision.HIGHEST
_SIGN = np.int32(-(2**31))


def _bit_const(bit: int) -> np.int32:
    return np.int32(np.uint32(1 << bit))


def _attn_body(x_ref, w_ref, adj_ref, out_ref):
    xb = x_ref[0]
    q = jnp.dot(xb, w_ref[0, 0], preferred_element_type=jnp.float32,
                precision=_HIGHEST)
    k = jnp.dot(xb, w_ref[1, 0], preferred_element_type=jnp.float32,
                precision=_HIGHEST)
    v = jnp.dot(xb, w_ref[2, 0], preferred_element_type=jnp.float32,
                precision=_HIGHEST)

    a = adj_ref[0]
    bits = jax.lax.bitcast_convert_type(a, jnp.int32)
    ikey = jnp.where(bits < 0, bits ^ np.int32(0x7FFFFFFF), bits)

    prefix_u = jnp.zeros((_T, 1), jnp.int32)
    for bit in range(31, -1, -1):
        cand_u = prefix_u | _bit_const(bit)
        cand_s = cand_u ^ _SIGN
        cnt = jnp.sum((ikey >= cand_s).astype(jnp.float32), axis=-1,
                      keepdims=True)
        prefix_u = jnp.where(cnt >= float(_TOPK), cand_u, prefix_u)
    thr_s = prefix_u ^ _SIGN

    gt = ikey > thr_s
    eq = ikey == thr_s
    cnt_gt = jnp.sum(gt.astype(jnp.float32), axis=-1, keepdims=True)
    rows = jax.lax.broadcasted_iota(jnp.int32, (_T, _T), 0)
    cols = jax.lax.broadcasted_iota(jnp.int32, (_T, _T), 1)
    upper_tri = (rows <= cols).astype(jnp.float32)
    eqf = eq.astype(jnp.float32)
    incl = jnp.dot(eqf, upper_tri, preferred_element_type=jnp.float32,
                   precision=_HIGHEST)
    need = float(_TOPK) - cnt_gt
    sel = gt | (eq & (incl <= need))

    scale = float(_DIM) ** (-0.5)
    s = jax.lax.dot_general(q * scale, k, (((1,), (1,)), ((), ())),
                            preferred_element_type=jnp.float32,
                            precision=_HIGHEST)
    s = jnp.where(sel, s, -jnp.inf)
    m = jnp.max(s, axis=-1, keepdims=True)
    p = jnp.exp(s - m)
    p = p / jnp.sum(p, axis=-1, keepdims=True)
    o = jnp.dot(p, v, preferred_element_type=jnp.float32,
                precision=_HIGHEST)
    out_ref[0, 0] = jax.nn.gelu(o)


@jax.jit
def kernel(x, adj, rep_adj_dis, Wqkv):
    del rep_adj_dis
    b, t, dim = x.shape
    h = _HEADS
    grid = (b * h,)
    w = Wqkv.reshape(dim, 3, h, _HD).transpose(1, 2, 0, 3)

    out = pl.pallas_call(
        _attn_body,
        grid=grid,
        in_specs=[
            pl.BlockSpec((1, t, dim), lambda g: (g // _HEADS, 0, 0)),
            pl.BlockSpec((3, 1, dim, _HD), lambda g: (0, g % _HEADS, 0, 0)),
            pl.BlockSpec((1, t, t), lambda g: (g, 0, 0)),
        ],
        out_specs=pl.BlockSpec((1, 1, t, _HD),
                               lambda g: (g // _HEADS, g % _HEADS, 0, 0)),
        out_shape=jax.ShapeDtypeStruct((b, h, t, _HD), jnp.float32),
    )(x, w, adj)
    return out.transpose(0, 2, 1, 3).reshape(b, t, dim)


# speedup vs baseline: 161.3684x; 1.7882x over previous
import jax
import jax.numpy as jnp
import numpy as np
from jax.experimental import pallas as pl

_DIM = 768
_HEADS = 12
_HD = _DIM // _HEADS
_TOPK = 64
_T = 256
_HG = 4
_NG = _HEADS // _HG
_B = 4

_DEFAULT = jax.lax.Precision.DEFAULT
_SIGN = np.int32(-(2**31))


def _bit_const(bit: int) -> np.int32:
    return np.int32(np.uint32(1 << bit))


def _dot(a, b):
    return jnp.dot(a, b, preferred_element_type=jnp.float32,
                   precision=_DEFAULT)


def _attn_body(x_ref, w_ref, adj_ref, out_ref):
    hg = pl.program_id(0) // _B
    xb = x_ref[0]
    qa = _dot(xb, w_ref[0, hg])
    ka = _dot(xb, w_ref[1, hg])
    va = _dot(xb, w_ref[2, hg])

    a = adj_ref[...]
    bits = jax.lax.bitcast_convert_type(a, jnp.int32)
    ikey = jnp.where(bits < 0, bits ^ np.int32(0x7FFFFFFF), bits)

    prefix_u = jnp.zeros((_HG, _T, 1), jnp.int32)
    for bit in range(31, -1, -1):
        cand_u = prefix_u | _bit_const(bit)
        cand_s = cand_u ^ _SIGN
        cnt = jnp.sum((ikey >= cand_s).astype(jnp.float32), axis=-1,
                      keepdims=True)
        prefix_u = jnp.where(cnt >= float(_TOPK), cand_u, prefix_u)
    thr_s = prefix_u ^ _SIGN

    gt = ikey > thr_s
    eq = ikey == thr_s
    cnt_gt = jnp.sum(gt.astype(jnp.float32), axis=-1, keepdims=True)
    need = float(_TOPK) - cnt_gt

    rows = jax.lax.broadcasted_iota(jnp.int32, (_T, _T), 0)
    cols = jax.lax.broadcasted_iota(jnp.int32, (_T, _T), 1)
    upper_tri = (rows <= cols).astype(jnp.float32)

    scale = float(_DIM) ** (-0.5)
    neg_inf = jnp.float32(-jnp.inf)
    for i in range(_HG):
        eqf = eq[i].astype(jnp.float32)
        incl = _dot(eqf, upper_tri)
        sel = gt[i] | (eq[i] & (incl <= need[i]))

        q = qa[:, _HD * i:_HD * (i + 1)]
        k = ka[:, _HD * i:_HD * (i + 1)]
        v = va[:, _HD * i:_HD * (i + 1)]
        s = jax.lax.dot_general(q * scale, k, (((1,), (1,)), ((), ())),
                                preferred_element_type=jnp.float32,
                                precision=_DEFAULT)
        s = jnp.where(sel, s, neg_inf)
        m = jnp.max(s, axis=-1, keepdims=True)
        p = jnp.exp(s - m)
        p = p / jnp.sum(p, axis=-1, keepdims=True)
        o = _dot(p, v)
        out_ref[0, i] = jax.nn.gelu(o)


@jax.jit
def kernel(x, adj, rep_adj_dis, Wqkv):
    del rep_adj_dis
    b, t, dim = x.shape
    w = Wqkv.reshape(dim, 3, _NG, _HG * _HD).transpose(1, 2, 0, 3)

    out = pl.pallas_call(
        _attn_body,
        grid=(_NG * b,),
        in_specs=[
            pl.BlockSpec((1, t, dim), lambda g: (g % _B, 0, 0)),
            pl.BlockSpec((3, _NG, dim, _HG * _HD), lambda g: (0, 0, 0, 0)),
            pl.BlockSpec((_HG, t, t),
                         lambda g: (_NG * (g % _B) + g // _B, 0, 0)),
        ],
        out_specs=pl.BlockSpec((1, _HG, t, _HD),
                               lambda g: (g % _B, g // _B, 0, 0)),
        out_shape=jax.ShapeDtypeStruct((b, _HEADS, t, _HD), jnp.float32),
    )(x, w, adj)
    return out.transpose(0, 2, 1, 3).reshape(b, t, dim)


# speedup vs baseline: 197.1795x; 1.2219x over previous
import jax
import jax.numpy as jnp
import numpy as np
from jax.experimental import pallas as pl

_DIM = 768
_HEADS = 12
_HD = _DIM // _HEADS
_TOPK = 64
_T = 256
_HG = 4
_NG = _HEADS // _HG
_B = 4

_DEFAULT = jax.lax.Precision.DEFAULT
_SIGN = np.int32(-(2**31))


def _bit_const(bit: int) -> np.int32:
    return np.int32(np.uint32(1 << bit))


def _dot(a, b):
    return jnp.dot(a, b, preferred_element_type=jnp.float32,
                   precision=_DEFAULT)


def _attn_body(x_ref, wq_ref, wk_ref, wv_ref, adjt_ref, out_ref):
    hg = pl.program_id(0) % _NG
    xb = x_ref[0]
    qa = _dot(xb, wq_ref[...])
    ka = _dot(xb, wk_ref[...])
    va = _dot(xb, wv_ref[...])

    at = adjt_ref[...]
    bits = jax.lax.bitcast_convert_type(at, jnp.int32)
    ikey = jnp.where(bits < 0, bits ^ np.int32(0x7FFFFFFF), bits)

    prefix_u = jnp.zeros((_HG, 1, _T), jnp.int32)
    for bit in range(31, -1, -1):
        cand_u = prefix_u | _bit_const(bit)
        cand_s = cand_u ^ _SIGN
        cnt = jnp.sum((ikey >= cand_s).astype(jnp.float32), axis=1,
                      keepdims=True)
        prefix_u = jnp.where(cnt >= float(_TOPK), cand_u, prefix_u)
    thr_s = prefix_u ^ _SIGN

    gt = ikey > thr_s
    eq = ikey == thr_s
    cnt_gt = jnp.sum(gt.astype(jnp.float32), axis=1, keepdims=True)
    need = float(_TOPK) - cnt_gt

    rows = jax.lax.broadcasted_iota(jnp.int32, (_T, _T), 0)
    cols = jax.lax.broadcasted_iota(jnp.int32, (_T, _T), 1)
    lower_tri = (rows >= cols).astype(jnp.float32)

    scale = float(_DIM) ** (-0.5)
    neg_inf = jnp.float32(-jnp.inf)
    outs = []
    for i in range(_HG):
        eqf = eq[i].astype(jnp.float32)
        incl = _dot(lower_tri, eqf)
        sel = gt[i] | (eq[i] & (incl <= need[i]))

        q = qa[:, _HD * i:_HD * (i + 1)]
        k = ka[:, _HD * i:_HD * (i + 1)]
        v = va[:, _HD * i:_HD * (i + 1)]
        st = jax.lax.dot_general(k, q * scale, (((1,), (1,)), ((), ())),
                                 preferred_element_type=jnp.float32,
                                 precision=_DEFAULT)
        st = jnp.where(sel, st, neg_inf)
        m = jnp.max(st, axis=0, keepdims=True)
        p = jnp.exp(st - m)
        p = p / jnp.sum(p, axis=0, keepdims=True)
        o = jax.lax.dot_general(p, v, (((0,), (0,)), ((), ())),
                                preferred_element_type=jnp.float32,
                                precision=_DEFAULT)
        outs.append(jax.nn.gelu(o))

    oo = jnp.concatenate(outs, axis=1)
    for c in range(_NG):
        @pl.when(hg == c)
        def _store(oo=oo, c=c):
            out_ref[0, :, _HG * _HD * c:_HG * _HD * (c + 1)] = oo


@jax.jit
def kernel(x, adj, rep_adj_dis, Wqkv):
    del rep_adj_dis
    b, t, dim = x.shape
    cw = _HG * _HD
    adjt = adj.transpose(0, 2, 1)

    out = pl.pallas_call(
        _attn_body,
        grid=(b * _NG,),
        in_specs=[
            pl.BlockSpec((1, t, dim), lambda g: (g // _NG, 0, 0)),
            pl.BlockSpec((dim, cw), lambda g: (0, g % _NG)),
            pl.BlockSpec((dim, cw), lambda g: (0, _NG + g % _NG)),
            pl.BlockSpec((dim, cw), lambda g: (0, 2 * _NG + g % _NG)),
            pl.BlockSpec((_HG, t, t), lambda g: (g, 0, 0)),
        ],
        out_specs=pl.BlockSpec((1, t, dim), lambda g: (g // _NG, 0, 0)),
        out_shape=jax.ShapeDtypeStruct((b, t, dim), jnp.float32),
    )(x, Wqkv, Wqkv, Wqkv, adjt)
    return out


# speedup vs baseline: 275.5650x; 1.3975x over previous
import jax
import jax.numpy as jnp
import numpy as np
from jax.experimental import pallas as pl

_DIM = 768
_HEADS = 12
_HD = _DIM // _HEADS
_TOPK = 64
_T = 256
_HG = 4
_NG = _HEADS // _HG
_B = 4

_DEFAULT = jax.lax.Precision.DEFAULT
_SIGN = np.int32(-(2**31))


def _bit_const(bit: int) -> np.int32:
    return np.int32(np.uint32(1 << bit))


def _dot(a, b):
    return jnp.dot(a, b, preferred_element_type=jnp.float32,
                   precision=_DEFAULT)


def _attn_body(x_ref, wq_ref, wk_ref, wv_ref, adj_ref, out_ref):
    hg = pl.program_id(0) % _NG
    xb = x_ref[0]
    qa = _dot(xb, wq_ref[...])
    ka = _dot(xb, wk_ref[...])
    va = _dot(xb, wv_ref[...])

    at = jnp.swapaxes(adj_ref[...], 1, 2)
    bits = jax.lax.bitcast_convert_type(at, jnp.int32)
    ikey = jnp.where(bits < 0, bits ^ np.int32(0x7FFFFFFF), bits)

    prefix_u = jnp.zeros((_HG, 1, _T), jnp.int32)
    for bit in range(31, -1, -1):
        cand_u = prefix_u | _bit_const(bit)
        cand_s = cand_u ^ _SIGN
        cnt = jnp.sum((ikey >= cand_s).astype(jnp.float32), axis=1,
                      keepdims=True)
        prefix_u = jnp.where(cnt >= float(_TOPK), cand_u, prefix_u)
    thr_s = prefix_u ^ _SIGN

    gt = ikey > thr_s
    eq = ikey == thr_s
    cnt_gt = jnp.sum(gt.astype(jnp.float32), axis=1, keepdims=True)
    need = float(_TOPK) - cnt_gt

    rows = jax.lax.broadcasted_iota(jnp.int32, (_T, _T), 0)
    cols = jax.lax.broadcasted_iota(jnp.int32, (_T, _T), 1)
    lower_tri = (rows >= cols).astype(jnp.float32)

    scale = float(_DIM) ** (-0.5)
    neg_inf = jnp.float32(-jnp.inf)
    outs = []
    for i in range(_HG):
        eqf = eq[i].astype(jnp.float32)
        incl = _dot(lower_tri, eqf)
        sel = gt[i] | (eq[i] & (incl <= need[i]))

        q = qa[:, _HD * i:_HD * (i + 1)]
        k = ka[:, _HD * i:_HD * (i + 1)]
        v = va[:, _HD * i:_HD * (i + 1)]
        st = jax.lax.dot_general(k, q * scale, (((1,), (1,)), ((), ())),
                                 preferred_element_type=jnp.float32,
                                 precision=_DEFAULT)
        st = jnp.where(sel, st, neg_inf)
        m = jnp.max(st, axis=0, keepdims=True)
        p = jnp.exp(st - m)
        p = p / jnp.sum(p, axis=0, keepdims=True)
        o = jax.lax.dot_general(p, v, (((0,), (0,)), ((), ())),
                                preferred_element_type=jnp.float32,
                                precision=_DEFAULT)
        outs.append(jax.nn.gelu(o))

    oo = jnp.concatenate(outs, axis=1)
    for c in range(_NG):
        @pl.when(hg == c)
        def _store(oo=oo, c=c):
            out_ref[0, :, _HG * _HD * c:_HG * _HD * (c + 1)] = oo


@jax.jit
def kernel(x, adj, rep_adj_dis, Wqkv):
    del rep_adj_dis
    b, t, dim = x.shape
    cw = _HG * _HD

    out = pl.pallas_call(
        _attn_body,
        grid=(b * _NG,),
        in_specs=[
            pl.BlockSpec((1, t, dim), lambda g: (g // _NG, 0, 0)),
            pl.BlockSpec((dim, cw), lambda g: (0, g % _NG)),
            pl.BlockSpec((dim, cw), lambda g: (0, _NG + g % _NG)),
            pl.BlockSpec((dim, cw), lambda g: (0, 2 * _NG + g % _NG)),
            pl.BlockSpec((_HG, t, t), lambda g: (g, 0, 0)),
        ],
        out_specs=pl.BlockSpec((1, t, dim), lambda g: (g // _NG, 0, 0)),
        out_shape=jax.ShapeDtypeStruct((b, t, dim), jnp.float32),
    )(x, Wqkv, Wqkv, Wqkv, adj)
    return out


# speedup vs baseline: 418.7086x; 1.5195x over previous
import jax
import jax.numpy as jnp
import numpy as np
from jax.experimental import pallas as pl

_DIM = 768
_HEADS = 12
_HD = _DIM // _HEADS
_TOPK = 64
_T = 256

_DEFAULT = jax.lax.Precision.DEFAULT
_SIGN16 = np.int16(np.uint16(0x8000))
_TOPK16 = np.int16(_TOPK)


def _bit_const16(bit: int) -> np.int16:
    return np.int16(np.uint16(1 << bit))


def _dot(a, b):
    return jnp.dot(a, b, preferred_element_type=jnp.float32,
                   precision=_DEFAULT)


def _count_sublanes(mask):
    s = mask.astype(jnp.int16)
    j = s.shape[1]
    while j > 1:
        j //= 2
        s = s[:, :j, :] + s[:, j:, :]
    return s


def _attn_body(x_ref, wq_ref, wk_ref, wv_ref, adj_ref, out_ref):
    xb = x_ref[0]
    qa = _dot(xb, wq_ref[...])
    ka = _dot(xb, wk_ref[...])
    va = _dot(xb, wv_ref[...])

    at = jnp.swapaxes(adj_ref[...], 1, 2)
    bits = jax.lax.bitcast_convert_type(at, jnp.int32)
    ikey = jnp.where(bits < 0, bits ^ np.int32(0x7FFFFFFF), bits)

    hi16 = (ikey >> 16).astype(jnp.int16)
    prefix_u = jnp.zeros((_HEADS, 1, _T), jnp.int16)
    for bit in range(15, -1, -1):
        cand_u = prefix_u | _bit_const16(bit)
        cand_s = cand_u ^ _SIGN16
        cnt = _count_sublanes(hi16 >= cand_s)
        take = cnt >= _TOPK16
        prefix_u = jnp.where(take, cand_u, prefix_u)
    thr_hi = prefix_u ^ _SIGN16

    lo16 = ikey.astype(jnp.int16) ^ _SIGN16
    lkey = jnp.where(hi16 > thr_hi, np.int16(32767),
                     jnp.where(hi16 == thr_hi, lo16, np.int16(-32768)))
    cnt_gei = _count_sublanes(hi16 >= thr_hi)
    prefix_u = jnp.zeros((_HEADS, 1, _T), jnp.int16)
    for bit in range(15, -1, -1):
        cand_u = prefix_u | _bit_const16(bit)
        cand_s = cand_u ^ _SIGN16
        cnt = _count_sublanes(lkey >= cand_s)
        take = cnt >= _TOPK16
        prefix_u = jnp.where(take, cand_u, prefix_u)
        cnt_gei = jnp.where(take, cnt, cnt_gei)
    thr_s = ((thr_hi.astype(jnp.int32) << 16)
             | (prefix_u.astype(jnp.int32) & np.int32(0xFFFF)))
    cnt_ge = cnt_gei.astype(jnp.float32)

    rows = jax.lax.broadcasted_iota(jnp.int32, (_T, _T), 0)
    cols = jax.lax.broadcasted_iota(jnp.int32, (_T, _T), 1)
    lower_tri = (rows >= cols).astype(jnp.float32)

    scale = float(_DIM) ** (-0.5)
    neg_inf = jnp.float32(-jnp.inf)
    outs = []
    for i in range(_HEADS):
        ge = ikey[i] >= thr_s[i]
        eq = ikey[i] == thr_s[i]
        eqf = eq.astype(jnp.float32)
        incl = _dot(lower_tri, eqf)
        need = float(_TOPK) - cnt_ge[i] + incl[_T - 1:_T, :]
        sel = ge & ((incl <= need) | ~eq)

        q = qa[:, _HD * i:_HD * (i + 1)]
        k = ka[:, _HD * i:_HD * (i + 1)]
        v = va[:, _HD * i:_HD * (i + 1)]
        st = jax.lax.dot_general(k, q * scale, (((1,), (1,)), ((), ())),
                                 preferred_element_type=jnp.float32,
                                 precision=_DEFAULT)
        st = jnp.where(sel, st, neg_inf)
        m = jnp.max(st, axis=0, keepdims=True)
        p = jnp.exp(st - m)
        p = p / jnp.sum(p, axis=0, keepdims=True)
        o = jax.lax.dot_general(p, v, (((0,), (0,)), ((), ())),
                                preferred_element_type=jnp.float32,
                                precision=_DEFAULT)
        outs.append(jax.nn.gelu(o))

    out_ref[0] = jnp.concatenate(outs, axis=1)


@jax.jit
def kernel(x, adj, rep_adj_dis, Wqkv):
    del rep_adj_dis
    b, t, dim = x.shape

    out = pl.pallas_call(
        _attn_body,
        grid=(b,),
        in_specs=[
            pl.BlockSpec((1, t, dim), lambda g: (g, 0, 0)),
            pl.BlockSpec((dim, dim), lambda g: (0, 0)),
            pl.BlockSpec((dim, dim), lambda g: (0, 1)),
            pl.BlockSpec((dim, dim), lambda g: (0, 2)),
            pl.BlockSpec((_HEADS, t, t), lambda g: (g, 0, 0)),
        ],
        out_specs=pl.BlockSpec((1, t, dim), lambda g: (g, 0, 0)),
        out_shape=jax.ShapeDtypeStruct((b, t, dim), jnp.float32),
    )(x, Wqkv, Wqkv, Wqkv, adj)
    return out
